# baseline (speedup 1.0000x reference)
import functools
import jax
import jax.numpy as jnp
from jax import lax
from jax.experimental import pallas as pl
from jax.experimental.pallas import tpu as pltpu
from jax.experimental.pallas import tpu_sc as plsc

E = 16
D = 2048
NW = 32
TPW = 512
L = 16
NEG_INF = float("-inf")


def _matmul_body(x_ref, w_ref, lt_ref):
    x = x_ref[...]
    w = w_ref[...]
    lt = jax.lax.dot_general(w, x, (((0,), (1,)), ((), ())),
                             preferred_element_type=jnp.float32)
    lt_ref[...] = lt[None]


def _tc_logits(x2, W, N, tpw, off):
    return pl.pallas_call(
        _matmul_body,
        grid=(N // tpw,),
        in_specs=[
            pl.BlockSpec((tpw, D), lambda i: (i + off, 0)),
            pl.BlockSpec((D, E), lambda i: (0, 0)),
        ],
        out_specs=pl.BlockSpec((1, E, tpw), lambda i: (i, 0, 0)),
        out_shape=jax.ShapeDtypeStruct((N // tpw, E, tpw), jnp.float32),
    )(x2, W)


def _sc_route_body(tpw, lt_hbm, wT_hbm, tw1_hbm, tw2_hbm, te1_hbm, te2_hbm,
                   lt_v, wT_v, tw1_v, tw2_v, te1_v, te2_v):
    c = lax.axis_index("c")
    s = lax.axis_index("s")
    wid = s * 2 + c
    pltpu.sync_copy(lt_hbm.at[wid], lt_v)

    def group(t0, _):
        sl = pl.ds(t0 * L, L)
        rows = [lt_v[e, sl] for e in range(E)]
        m = rows[0]
        i1 = jnp.zeros((L,), jnp.int32)
        for e in range(1, E):
            p = rows[e] > m
            m = jnp.where(p, rows[e], m)
            i1 = jnp.where(p, e, i1)
        m2 = jnp.full((L,), NEG_INF, jnp.float32)
        i2 = jnp.zeros((L,), jnp.int32)
        ssum = jnp.zeros((L,), jnp.float32)
        exs = []
        for e in range(E):
            ex = jnp.exp(rows[e] - m)
            exs.append(ex)
            ssum = ssum + ex
            me = jnp.where(i1 == e, NEG_INF, rows[e])
            p2 = me > m2
            m2 = jnp.where(p2, me, m2)
            i2 = jnp.where(p2, e, i2)
        rs = 1.0 / ssum
        for e in range(E):
            wT_v[e, sl] = exs[e] * rs
        e2 = jnp.exp(m2 - m)
        r = 1.0 / (1.0 + e2)
        tw1_v[sl] = r
        tw2_v[sl] = e2 * r
        te1_v[sl] = i1
        te2_v[sl] = i2
        return 0

    lax.fori_loop(0, tpw // L, group, 0)

    pltpu.sync_copy(wT_v, wT_hbm.at[wid])
    pltpu.sync_copy(tw1_v, tw1_hbm.at[wid])
    pltpu.sync_copy(tw2_v, tw2_hbm.at[wid])
    pltpu.sync_copy(te1_v, te1_hbm.at[wid])
    pltpu.sync_copy(te2_v, te2_hbm.at[wid])


def _sc_route(lt, tpw):
    mesh = plsc.VectorSubcoreMesh(core_axis_name="c", subcore_axis_name="s")
    f = functools.partial(
        pl.kernel, mesh=mesh,
        out_type=[
            jax.ShapeDtypeStruct((NW, E, tpw), jnp.float32),
            jax.ShapeDtypeStruct((NW, tpw), jnp.float32),
            jax.ShapeDtypeStruct((NW, tpw), jnp.float32),
            jax.ShapeDtypeStruct((NW, tpw), jnp.int32),
            jax.ShapeDtypeStruct((NW, tpw), jnp.int32),
        ],
        scratch_types=[
            pltpu.VMEM((E, tpw), jnp.float32),
            pltpu.VMEM((E, tpw), jnp.float32),
            pltpu.VMEM((tpw,), jnp.float32),
            pltpu.VMEM((tpw,), jnp.float32),
            pltpu.VMEM((tpw,), jnp.int32),
            pltpu.VMEM((tpw,), jnp.int32),
        ],
    )(functools.partial(_sc_route_body, tpw))
    return f(lt)


def kernel(x, W):
    B, S, _ = x.shape
    N = B * S
    x2 = x.reshape(N, D)
    H = N // 2
    nblk = H // (TPW // 2)
    lt0 = _tc_logits(x2, W, H, TPW // 2, 0)
    lt1 = _tc_logits(x2, W, H, TPW // 2, nblk)
    parts = [_sc_route(lt0, TPW // 2), _sc_route(lt1, TPW // 2)]
    wT = jnp.concatenate([p[0] for p in parts], axis=0)
    tw1 = jnp.concatenate([p[1] for p in parts], axis=0)
    tw2 = jnp.concatenate([p[2] for p in parts], axis=0)
    te1 = jnp.concatenate([p[3] for p in parts], axis=0)
    te2 = jnp.concatenate([p[4] for p in parts], axis=0)
    weights = wT.transpose(0, 2, 1).reshape(B, S, E)
    topw = jnp.stack([tw1.reshape(N), tw2.reshape(N)], axis=-1).reshape(B, S, 2)
    tope = jnp.stack([te1.reshape(N), te2.reshape(N)], axis=-1).reshape(B, S, 2)
    return weights, topw, tope


# speedup vs baseline: 1.2080x; 1.2080x over previous
import functools
import jax
import jax.numpy as jnp
from jax import lax
from jax.experimental import pallas as pl
from jax.experimental.pallas import tpu as pltpu
from jax.experimental.pallas import tpu_sc as plsc

E = 16
D = 2048
NW = 32
TPW = 512
L = 16
NEG_INF = float("-inf")


def _matmul_body(x_ref, w_ref, lt_ref):
    x = x_ref[...]
    w = w_ref[...]
    lt = jax.lax.dot_general(w, x, (((0,), (1,)), ((), ())),
                             preferred_element_type=jnp.float32)
    lt_ref[...] = lt[None]


def _tc_logits(x2, W, N, tpw, off):
    return pl.pallas_call(
        _matmul_body,
        grid=(N // tpw,),
        in_specs=[
            pl.BlockSpec((tpw, D), lambda i: (i + off, 0)),
            pl.BlockSpec((D, E), lambda i: (0, 0)),
        ],
        out_specs=pl.BlockSpec((1, E, tpw), lambda i: (i, 0, 0)),
        out_shape=jax.ShapeDtypeStruct((N // tpw, E, tpw), jnp.float32),
    )(x2, W)


def _sc_route_body(tpw, lt_hbm, wT_hbm, tw1_hbm, tw2_hbm, te1_hbm, te2_hbm,
                   lt_v, wT_v, tw1_v, tw2_v, te1_v, te2_v):
    c = lax.axis_index("c")
    s = lax.axis_index("s")
    wid = s * 2 + c
    pltpu.sync_copy(lt_hbm.at[wid], lt_v)

    def group(t0, _):
        sl = pl.ds(t0 * L, L)
        rows = [lt_v[e, sl] for e in range(E)]
        m = rows[0]
        i1 = jnp.zeros((L,), jnp.int32)
        for e in range(1, E):
            p = rows[e] > m
            m = jnp.where(p, rows[e], m)
            i1 = jnp.where(p, e, i1)
        m2 = jnp.full((L,), NEG_INF, jnp.float32)
        i2 = jnp.zeros((L,), jnp.int32)
        ssum = jnp.zeros((L,), jnp.float32)
        exs = []
        for e in range(E):
            ex = jnp.exp(rows[e] - m)
            exs.append(ex)
            ssum = ssum + ex
            me = jnp.where(i1 == e, NEG_INF, rows[e])
            p2 = me > m2
            m2 = jnp.where(p2, me, m2)
            i2 = jnp.where(p2, e, i2)
        rs = 1.0 / ssum
        for e in range(E):
            wT_v[e, sl] = exs[e] * rs
        e2 = jnp.exp(m2 - m)
        r = 1.0 / (1.0 + e2)
        tw1_v[sl] = r
        tw2_v[sl] = e2 * r
        te1_v[sl] = i1
        te2_v[sl] = i2
        return 0

    lax.fori_loop(0, tpw // L, group, 0)

    pltpu.sync_copy(wT_v, wT_hbm.at[wid])
    pltpu.sync_copy(tw1_v, tw1_hbm.at[wid])
    pltpu.sync_copy(tw2_v, tw2_hbm.at[wid])
    pltpu.sync_copy(te1_v, te1_hbm.at[wid])
    pltpu.sync_copy(te2_v, te2_hbm.at[wid])


def _sc_route(lt, tpw):
    mesh = plsc.VectorSubcoreMesh(core_axis_name="c", subcore_axis_name="s")
    f = functools.partial(
        pl.kernel, mesh=mesh,
        out_type=[
            jax.ShapeDtypeStruct((NW, E, tpw), jnp.float32),
            jax.ShapeDtypeStruct((NW, tpw), jnp.float32),
            jax.ShapeDtypeStruct((NW, tpw), jnp.float32),
            jax.ShapeDtypeStruct((NW, tpw), jnp.int32),
            jax.ShapeDtypeStruct((NW, tpw), jnp.int32),
        ],
        scratch_types=[
            pltpu.VMEM((E, tpw), jnp.float32),
            pltpu.VMEM((E, tpw), jnp.float32),
            pltpu.VMEM((tpw,), jnp.float32),
            pltpu.VMEM((tpw,), jnp.float32),
            pltpu.VMEM((tpw,), jnp.int32),
            pltpu.VMEM((tpw,), jnp.int32),
        ],
    )(functools.partial(_sc_route_body, tpw))
    return f(lt)


def kernel(x, W):
    B, S, _ = x.shape
    N = B * S
    x2 = x.reshape(N, D)
    lt = _tc_logits(x2, W, N, TPW, 0)
    wT, tw1, tw2, te1, te2 = _sc_route(lt, TPW)
    weights = wT.transpose(0, 2, 1).reshape(B, S, E)
    topw = jnp.stack([tw1.reshape(N), tw2.reshape(N)], axis=-1).reshape(B, S, 2)
    tope = jnp.stack([te1.reshape(N), te2.reshape(N)], axis=-1).reshape(B, S, 2)
    return weights, topw, tope
